# baseline (speedup 1.0000x reference)
import functools

import jax
import jax.numpy as jnp
from jax import lax
from jax.experimental import pallas as pl
from jax.experimental.pallas import tpu as pltpu
from jax.experimental.pallas import tpu_sc as plsc

VOCAB = 100000
SEQ = 50
EMB = 64
H1 = 1024
H2 = 256
BATCH = 4096
N_IDX = BATCH * SEQ
IN1 = SEQ * EMB

GATHER_WIN = 128

NCHUNK = 4
CHUNK_B = BATCH // NCHUNK
CHUNK_IDX = CHUNK_B * SEQ

BATCH_BLK = 1024


def _sc_gather(emb, idx_flat):
    mesh = plsc.VectorSubcoreMesh(core_axis_name="c", subcore_axis_name="s")
    grid = CHUNK_IDX // GATHER_WIN

    @functools.partial(
        pl.kernel,
        out_type=jax.ShapeDtypeStruct((CHUNK_IDX, EMB), jnp.float32),
        mesh=mesh,
        compiler_params=pltpu.CompilerParams(use_tc_tiling_on_sc=False),
    )
    def gather_kernel(emb_hbm, idx_hbm, out_hbm):
        def body(idx_vmem, out_vmem):
            pltpu.sync_copy(emb_hbm.at[idx_vmem], out_vmem)

        pltpu.emit_pipeline(
            body,
            grid=(grid,),
            in_specs=[pl.BlockSpec((GATHER_WIN,), index_map=lambda i: (i,))],
            out_specs=[pl.BlockSpec((GATHER_WIN, EMB), index_map=lambda i: (i, 0))],
            core_axis_name=("c", "s"),
            dimension_semantics=(pltpu.PARALLEL,),
        )(idx_hbm, out_hbm)

    return gather_kernel(emb, idx_flat)


def _collapse_weights(W1, b1, W2, b2):
    def body(w1_ref, b1_ref, w2_ref, b2_ref, wc_ref, bc_ref):
        wc_ref[...] = jnp.dot(
            w1_ref[...], w2_ref[...],
            preferred_element_type=jnp.float32,
            precision=lax.Precision.DEFAULT,
        )
        bc_ref[...] = jnp.dot(
            b1_ref[...], w2_ref[...],
            preferred_element_type=jnp.float32,
            precision=lax.Precision.DEFAULT,
        ) + b2_ref[...]

    return pl.pallas_call(
        body,
        out_shape=(
            jax.ShapeDtypeStruct((IN1, H2), jnp.float32),
            jax.ShapeDtypeStruct((1, H2), jnp.float32),
        ),
    )(W1, b1.reshape(1, H1), W2, b2.reshape(1, H2))


def _dense_sigmoid(x, wc, bc):
    def body(x_ref, wc_ref, bc_ref, o_ref):
        acc = jnp.dot(
            x_ref[...].astype(jnp.bfloat16),
            wc_ref[...].astype(jnp.bfloat16),
            preferred_element_type=jnp.float32,
        )
        o_ref[...] = jax.nn.sigmoid(acc + bc_ref[...])

    return pl.pallas_call(
        body,
        grid=(CHUNK_B // BATCH_BLK,),
        in_specs=[
            pl.BlockSpec((BATCH_BLK, IN1), lambda i: (i, 0)),
            pl.BlockSpec((IN1, H2), lambda i: (0, 0)),
            pl.BlockSpec((1, H2), lambda i: (0, 0)),
        ],
        out_specs=pl.BlockSpec((BATCH_BLK, H2), lambda i: (i, 0)),
        out_shape=jax.ShapeDtypeStruct((CHUNK_B, H2), jnp.float32),
    )(x, wc, bc)


def kernel(sequences, emb, W1, b1, W2, b2):
    idx = sequences.reshape(-1).astype(jnp.int32)
    wc, bc = _collapse_weights(W1, b1, W2, b2)
    outs = []
    for c in range(NCHUNK):
        rows = _sc_gather(emb, idx[c * CHUNK_IDX:(c + 1) * CHUNK_IDX])
        x = rows.reshape(CHUNK_B, IN1)
        outs.append(_dense_sigmoid(x, wc, bc))
    return jnp.concatenate(outs, axis=0)


# speedup vs baseline: 1.0221x; 1.0221x over previous
import functools

import jax
import jax.numpy as jnp
from jax import lax
from jax.experimental import pallas as pl
from jax.experimental.pallas import tpu as pltpu
from jax.experimental.pallas import tpu_sc as plsc

VOCAB = 100000
SEQ = 50
EMB = 64
H1 = 1024
H2 = 256
BATCH = 4096
N_IDX = BATCH * SEQ
IN1 = SEQ * EMB

GATHER_WIN = 128

NCHUNK = 4
CHUNK_B = BATCH // NCHUNK
CHUNK_IDX = CHUNK_B * SEQ

BATCH_BLK = 512


def _sc_gather(emb, idx_flat):
    mesh = plsc.VectorSubcoreMesh(core_axis_name="c", subcore_axis_name="s")
    grid = CHUNK_IDX // GATHER_WIN

    @functools.partial(
        pl.kernel,
        out_type=jax.ShapeDtypeStruct((CHUNK_IDX, EMB), jnp.float32),
        mesh=mesh,
        compiler_params=pltpu.CompilerParams(use_tc_tiling_on_sc=False),
    )
    def gather_kernel(emb_hbm, idx_hbm, out_hbm):
        def body(idx_vmem, out_vmem):
            pltpu.sync_copy(emb_hbm.at[idx_vmem], out_vmem)

        pltpu.emit_pipeline(
            body,
            grid=(grid,),
            in_specs=[pl.BlockSpec((GATHER_WIN,), index_map=lambda i: (i,))],
            out_specs=[pl.BlockSpec((GATHER_WIN, EMB), index_map=lambda i: (i, 0))],
            core_axis_name=("c", "s"),
            dimension_semantics=(pltpu.PARALLEL,),
        )(idx_hbm, out_hbm)

    return gather_kernel(emb, idx_flat)


def _collapse_weights(W1, b1, W2, b2):
    def body(w1_ref, b1_ref, w2_ref, b2_ref, wc_ref, bc_ref):
        wc_ref[...] = jnp.dot(
            w1_ref[...], w2_ref[...],
            preferred_element_type=jnp.float32,
            precision=lax.Precision.DEFAULT,
        )
        bc_ref[...] = jnp.dot(
            b1_ref[...], w2_ref[...],
            preferred_element_type=jnp.float32,
            precision=lax.Precision.DEFAULT,
        ) + b2_ref[...]

    return pl.pallas_call(
        body,
        out_shape=(
            jax.ShapeDtypeStruct((IN1, H2), jnp.float32),
            jax.ShapeDtypeStruct((1, H2), jnp.float32),
        ),
    )(W1, b1.reshape(1, H1), W2, b2.reshape(1, H2))


def _dense_sigmoid(x, wc, bc):
    def body(x_ref, wc_ref, bc_ref, o_ref):
        acc = jnp.dot(
            x_ref[...].astype(jnp.bfloat16),
            wc_ref[...].astype(jnp.bfloat16),
            preferred_element_type=jnp.float32,
        )
        o_ref[...] = jax.nn.sigmoid(acc + bc_ref[...])

    return pl.pallas_call(
        body,
        grid=(CHUNK_B // BATCH_BLK,),
        in_specs=[
            pl.BlockSpec((BATCH_BLK, IN1), lambda i: (i, 0)),
            pl.BlockSpec((IN1, H2), lambda i: (0, 0)),
            pl.BlockSpec((1, H2), lambda i: (0, 0)),
        ],
        out_specs=pl.BlockSpec((BATCH_BLK, H2), lambda i: (i, 0)),
        out_shape=jax.ShapeDtypeStruct((CHUNK_B, H2), jnp.float32),
    )(x, wc, bc)


def kernel(sequences, emb, W1, b1, W2, b2):
    idx = sequences.reshape(-1).astype(jnp.int32)
    wc, bc = _collapse_weights(W1, b1, W2, b2)
    outs = []
    for c in range(NCHUNK):
        rows = _sc_gather(emb, idx[c * CHUNK_IDX:(c + 1) * CHUNK_IDX])
        x = rows.reshape(CHUNK_B, IN1)
        outs.append(_dense_sigmoid(x, wc, bc))
    return jnp.concatenate(outs, axis=0)
